# baseline (speedup 1.0000x reference)
import functools

import jax
import jax.numpy as jnp
from jax import lax
from jax.experimental import pallas as pl
from jax.experimental.pallas import tpu as pltpu
from jax.experimental.pallas import tpu_sc as plsc

B = 4096
D = 32
LANE = 128

_info = plsc.get_sparse_core_info()
_NC, _NS = _info.num_cores, _info.num_subcores
_NW = _NC * _NS
_BPW = B // _NW
_CH = 8

_mesh = plsc.VectorSubcoreMesh(core_axis_name="c", subcore_axis_name="s")


@functools.partial(
    pl.kernel,
    mesh=_mesh,
    out_type=[
        jax.ShapeDtypeStruct((D, B), jnp.float32),
        jax.ShapeDtypeStruct((D, B), jnp.float32),
    ],
    scratch_types=[
        pltpu.VMEM((_BPW,), jnp.int32),
        pltpu.VMEM((_BPW,), jnp.int32),
        pltpu.VMEM((2, _CH, D, LANE), jnp.float32),
        pltpu.VMEM((D, _BPW), jnp.float32),
        pltpu.VMEM((D, _BPW), jnp.float32),
        pltpu.SemaphoreType.DMA,
        pltpu.SemaphoreType.DMA,
    ],
    compiler_params=pltpu.CompilerParams(
        use_tc_tiling_on_sc=True, needs_layout_passes=False
    ),
)
def _sc_gather(u_hbm, i_hbm, utT_hbm, itT_hbm, wT_hbm, hT_hbm,
               u_vm, i_vm, slab_v, w_v, h_v, sem0, sem1):
    wid = lax.axis_index("s") * _NC + lax.axis_index("c")
    base = wid * _BPW
    pltpu.sync_copy(u_hbm.at[pl.ds(base, _BPW)], u_vm)
    pltpu.sync_copy(i_hbm.at[pl.ds(base, _BPW)], i_vm)

    rows0 = lax.iota(jnp.int32, 16)
    rows1 = rows0 + 16

    def gather_one(table_hbm, idx_vm, dst_v):
        nch = _BPW // _CH

        def fire(idx16, off, buf, sem):
            for k in range(_CH):
                idx = idx16[off + k]
                col0 = pl.multiple_of((idx >> 7) * LANE, LANE)
                pltpu.async_copy(
                    table_hbm.at[:, pl.ds(col0, LANE)],
                    slab_v.at[buf, k],
                    sem,
                )

        def drain_extract(idx16, off, c, buf, sem):
            for k in range(_CH):
                pltpu.make_async_copy(
                    table_hbm.at[:, pl.ds(0, LANE)],
                    slab_v.at[buf, k],
                    sem,
                ).wait()
            for k in range(_CH):
                idx = idx16[off + k]
                r = jnp.broadcast_to(idx & (LANE - 1), (16,))
                jcol = jnp.broadcast_to(c * _CH + k, (16,))
                g0 = plsc.load_gather(slab_v.at[buf, k], [rows0, r])
                g1 = plsc.load_gather(slab_v.at[buf, k], [rows1, r])
                plsc.store_scatter(dst_v, [rows0, jcol], g0)
                plsc.store_scatter(dst_v, [rows1, jcol], g1)

        fire(idx_vm[pl.ds(0, 16)], 0, 0, sem0)

        def body(c2, carry):
            c = c2 * 2
            idx16 = idx_vm[pl.ds(c2 * 2 * _CH, 2 * _CH)]
            fire(idx16, _CH, 1, sem1)
            drain_extract(idx16, 0, c, 0, sem0)

            @pl.when(c2 < nch // 2 - 1)
            def _fire_next():
                idx16n = idx_vm[pl.ds((c2 + 1) * 2 * _CH, 2 * _CH)]
                fire(idx16n, 0, 0, sem0)

            drain_extract(idx16, _CH, c + 1, 1, sem1)
            return carry

        lax.fori_loop(0, nch // 2, body, None)

    gather_one(utT_hbm, u_vm, w_v)
    gather_one(itT_hbm, i_vm, h_v)
    pltpu.sync_copy(w_v, wT_hbm.at[:, pl.ds(base, _BPW)])
    pltpu.sync_copy(h_v, hT_hbm.at[:, pl.ds(base, _BPW)])


_BM = 1024


def _mm_body(hT_ref, wT_ref, out_ref):
    out_ref[...] = lax.dot_general(
        hT_ref[...].astype(jnp.bfloat16), wT_ref[...].astype(jnp.bfloat16),
        (((0,), (0,)), ((), ())),
        preferred_element_type=jnp.float32,
    )


def _tc_matmul(hT, wT):
    return pl.pallas_call(
        _mm_body,
        grid=(B // _BM,),
        in_specs=[
            pl.BlockSpec((D, _BM), lambda m: (0, m)),
            pl.BlockSpec((D, B), lambda m: (0, 0)),
        ],
        out_specs=pl.BlockSpec((_BM, B), lambda m: (m, 0)),
        out_shape=jax.ShapeDtypeStruct((B, B), jnp.float32),
    )(hT, wT)


@jax.jit
def kernel(u, i, user_table, item_table):
    utT = user_table.T
    itT = item_table.T
    wT, hT = _sc_gather(u, i, utT, itT)
    return _tc_matmul(hT, wT)


# speedup vs baseline: 1.0175x; 1.0175x over previous
import functools

import jax
import jax.numpy as jnp
from jax import lax
from jax.experimental import pallas as pl
from jax.experimental.pallas import tpu as pltpu
from jax.experimental.pallas import tpu_sc as plsc

B = 4096
D = 32
LANE = 128

_info = plsc.get_sparse_core_info()
_NC, _NS = _info.num_cores, _info.num_subcores
_NW = _NC * _NS
_BPW = B // _NW
_CH = 8

_mesh = plsc.VectorSubcoreMesh(core_axis_name="c", subcore_axis_name="s")


@functools.partial(
    pl.kernel,
    mesh=_mesh,
    out_type=[
        jax.ShapeDtypeStruct((D, B), jnp.float32),
        jax.ShapeDtypeStruct((D, B), jnp.float32),
    ],
    scratch_types=[
        pltpu.VMEM((_BPW,), jnp.int32),
        pltpu.VMEM((_BPW,), jnp.int32),
        pltpu.VMEM((2, _CH, D, LANE), jnp.float32),
        pltpu.VMEM((D, _BPW), jnp.float32),
        pltpu.VMEM((D, _BPW), jnp.float32),
        pltpu.SemaphoreType.DMA,
        pltpu.SemaphoreType.DMA,
    ],
    compiler_params=pltpu.CompilerParams(
        use_tc_tiling_on_sc=True, needs_layout_passes=False
    ),
)
def _sc_gather(u_hbm, i_hbm, utT_hbm, itT_hbm, wT_hbm, hT_hbm,
               u_vm, i_vm, slab_v, w_v, h_v, sem0, sem1):
    wid = lax.axis_index("s") * _NC + lax.axis_index("c")
    base = wid * _BPW
    pltpu.sync_copy(u_hbm.at[pl.ds(base, _BPW)], u_vm)
    pltpu.sync_copy(i_hbm.at[pl.ds(base, _BPW)], i_vm)

    rows0 = lax.iota(jnp.int32, 16)
    rows1 = rows0 + 16

    def gather_one(table_hbm, idx_vm, dst_v):
        nch = _BPW // _CH

        def fire(idx16, off, buf, sem):
            for k in range(_CH):
                idx = idx16[off + k]
                col0 = pl.multiple_of((idx >> 7) * LANE, LANE)
                pltpu.async_copy(
                    table_hbm.at[:, pl.ds(col0, LANE)],
                    slab_v.at[buf, k],
                    sem,
                )

        def drain_extract(idx16, off, c, buf, sem):
            for k in range(_CH):
                pltpu.make_async_copy(
                    table_hbm.at[:, pl.ds(0, LANE)],
                    slab_v.at[buf, k],
                    sem,
                ).wait()
            for k in range(_CH):
                idx = idx16[off + k]
                r = jnp.broadcast_to(idx & (LANE - 1), (16,))
                jcol = jnp.broadcast_to(c * _CH + k, (16,))
                g0 = plsc.load_gather(slab_v.at[buf, k], [rows0, r])
                g1 = plsc.load_gather(slab_v.at[buf, k], [rows1, r])
                plsc.store_scatter(dst_v, [rows0, jcol], g0)
                plsc.store_scatter(dst_v, [rows1, jcol], g1)

        fire(idx_vm[pl.ds(0, 16)], 0, 0, sem0)

        def body(c2, carry):
            c = c2 * 2
            idx16 = idx_vm[pl.ds(c2 * 2 * _CH, 2 * _CH)]
            fire(idx16, _CH, 1, sem1)
            drain_extract(idx16, 0, c, 0, sem0)

            @pl.when(c2 < nch // 2 - 1)
            def _fire_next():
                idx16n = idx_vm[pl.ds((c2 + 1) * 2 * _CH, 2 * _CH)]
                fire(idx16n, 0, 0, sem0)

            drain_extract(idx16, _CH, c + 1, 1, sem1)
            return carry

        lax.fori_loop(0, nch // 2, body, None)

    gather_one(utT_hbm, u_vm, w_v)
    gather_one(itT_hbm, i_vm, h_v)
    pltpu.sync_copy(w_v, wT_hbm.at[:, pl.ds(base, _BPW)])
    pltpu.sync_copy(h_v, hT_hbm.at[:, pl.ds(base, _BPW)])


_BM = 512


def _mm_body(hT_ref, wT_ref, out_ref):
    out_ref[...] = lax.dot_general(
        hT_ref[...].astype(jnp.bfloat16), wT_ref[...].astype(jnp.bfloat16),
        (((0,), (0,)), ((), ())),
        preferred_element_type=jnp.float32,
    )


def _tc_matmul(hT, wT):
    return pl.pallas_call(
        _mm_body,
        grid=(B // _BM,),
        in_specs=[
            pl.BlockSpec((D, _BM), lambda m: (0, m)),
            pl.BlockSpec((D, B), lambda m: (0, 0)),
        ],
        out_specs=pl.BlockSpec((_BM, B), lambda m: (m, 0)),
        out_shape=jax.ShapeDtypeStruct((B, B), jnp.float32),
    )(hT, wT)


@jax.jit
def kernel(u, i, user_table, item_table):
    utT = user_table.T
    itT = item_table.T
    wT, hT = _sc_gather(u, i, utT, itT)
    return _tc_matmul(hT, wT)


# speedup vs baseline: 1.0316x; 1.0138x over previous
import functools

import jax
import jax.numpy as jnp
from jax import lax
from jax.experimental import pallas as pl
from jax.experimental.pallas import tpu as pltpu
from jax.experimental.pallas import tpu_sc as plsc

B = 4096
D = 32
LANE = 128

_info = plsc.get_sparse_core_info()
_NC, _NS = _info.num_cores, _info.num_subcores
_NW = _NC * _NS
_BPW = B // _NW
_CH = 8

_mesh = plsc.VectorSubcoreMesh(core_axis_name="c", subcore_axis_name="s")


@functools.partial(
    pl.kernel,
    mesh=_mesh,
    out_type=[
        jax.ShapeDtypeStruct((D, B), jnp.float32),
        jax.ShapeDtypeStruct((D, B), jnp.float32),
    ],
    scratch_types=[
        pltpu.VMEM((_BPW + 16,), jnp.int32),
        pltpu.VMEM((_BPW + 16,), jnp.int32),
        pltpu.VMEM((2, _CH, D, LANE), jnp.float32),
        pltpu.VMEM((D, _BPW), jnp.float32),
        pltpu.VMEM((D, _BPW), jnp.float32),
        pltpu.SemaphoreType.DMA,
        pltpu.SemaphoreType.DMA,
    ],
    compiler_params=pltpu.CompilerParams(
        use_tc_tiling_on_sc=True, needs_layout_passes=False
    ),
)
def _sc_gather(u_hbm, i_hbm, utT_hbm, itT_hbm, wT_hbm, hT_hbm,
               u_vm, i_vm, slab_v, w_v, h_v, sem0, sem1):
    wid = lax.axis_index("s") * _NC + lax.axis_index("c")
    base = wid * _BPW
    pltpu.sync_copy(u_hbm.at[pl.ds(base, _BPW)], u_vm.at[pl.ds(0, _BPW)])
    pltpu.sync_copy(i_hbm.at[pl.ds(base, _BPW)], i_vm.at[pl.ds(0, _BPW)])

    rows0 = lax.iota(jnp.int32, 16)
    rows1 = rows0 + 16
    nch = _BPW // _CH

    def fire(table_hbm, idx16, buf, sem):
        for k in range(_CH):
            idx = idx16[k]
            col0 = pl.multiple_of((idx >> 7) * LANE, LANE)
            pltpu.async_copy(
                table_hbm.at[:, pl.ds(col0, LANE)],
                slab_v.at[buf, k],
                sem,
            )

    def drain_extract(idx16, c, buf, sem, dst_v):
        for k in range(_CH):
            pltpu.make_async_copy(
                utT_hbm.at[:, pl.ds(0, LANE)],
                slab_v.at[buf, k],
                sem,
            ).wait()
        for k in range(_CH):
            idx = idx16[k]
            r = jnp.broadcast_to(idx & (LANE - 1), (16,))
            jcol = jnp.broadcast_to(c * _CH + k, (16,))
            g0 = plsc.load_gather(slab_v.at[buf, k], [rows0, r])
            g1 = plsc.load_gather(slab_v.at[buf, k], [rows1, r])
            plsc.store_scatter(dst_v, [rows0, jcol], g0)
            plsc.store_scatter(dst_v, [rows1, jcol], g1)

    fire(utT_hbm, u_vm[pl.ds(0, 16)], 0, sem0)

    def body(c, carry):
        u16 = u_vm[pl.ds(c * _CH, 16)]
        i16 = i_vm[pl.ds(c * _CH, 16)]
        fire(itT_hbm, i16, 1, sem1)
        drain_extract(u16, c, 0, sem0, w_v)

        @pl.when(c < nch - 1)
        def _fire_next():
            u16n = u_vm[pl.ds((c + 1) * _CH, 16)]
            fire(utT_hbm, u16n, 0, sem0)

        drain_extract(i16, c, 1, sem1, h_v)
        return carry

    lax.fori_loop(0, nch, body, None)
    pltpu.sync_copy(w_v, wT_hbm.at[:, pl.ds(base, _BPW)])
    pltpu.sync_copy(h_v, hT_hbm.at[:, pl.ds(base, _BPW)])


_BM = 512


def _mm_body(hT_ref, wT_ref, out_ref):
    out_ref[...] = lax.dot_general(
        hT_ref[...].astype(jnp.bfloat16), wT_ref[...].astype(jnp.bfloat16),
        (((0,), (0,)), ((), ())),
        preferred_element_type=jnp.float32,
    )


def _tc_matmul(hT, wT):
    return pl.pallas_call(
        _mm_body,
        grid=(B // _BM,),
        in_specs=[
            pl.BlockSpec((D, _BM), lambda m: (0, m)),
            pl.BlockSpec((D, B), lambda m: (0, 0)),
        ],
        out_specs=pl.BlockSpec((_BM, B), lambda m: (m, 0)),
        out_shape=jax.ShapeDtypeStruct((B, B), jnp.float32),
    )(hT, wT)


@jax.jit
def kernel(u, i, user_table, item_table):
    utT = user_table.T
    itT = item_table.T
    wT, hT = _sc_gather(u, i, utT, itT)
    return _tc_matmul(hT, wT)
